# baseline (speedup 1.0000x reference)
import functools

import jax
import jax.numpy as jnp
from jax import lax
from jax.experimental import pallas as pl
from jax.experimental.pallas import tpu as pltpu
from jax.experimental.pallas import tpu_sc as plsc

_SIZE = 16384
_DIM = 64
_NNEG = 5
_NW = 32
_PER_W = _SIZE // _NW
_CH = 128
_NCH = _PER_W // _CH


def _sc_scores_body(ct_hbm, ot_hbm, nt_hbm, out_hbm, c_v, o_v, n_v, s_v, isem, osem):
    wid = lax.axis_index("s") * 2 + lax.axis_index("c")
    base = wid * _PER_W

    def start_in(ci, buf):
        b0 = base + ci * _CH
        return (
            pltpu.async_copy(ct_hbm.at[:, pl.ds(b0, _CH)], c_v.at[buf], isem.at[buf, 0]),
            pltpu.async_copy(ot_hbm.at[:, pl.ds(b0, _CH)], o_v.at[buf], isem.at[buf, 1]),
            pltpu.async_copy(nt_hbm.at[:, :, pl.ds(b0, _CH)], n_v.at[buf], isem.at[buf, 2]),
        )

    pending = {0: start_in(0, 0)}
    out_pending = {}
    for ci in range(_NCH):
        buf = ci % 2
        for h in pending.pop(ci):
            h.wait()
        if ci + 1 < _NCH:
            pending[ci + 1] = start_in(ci + 1, (ci + 1) % 2)
        if ci >= 2 and (ci - 2) in out_pending:
            out_pending.pop(ci - 2).wait()

        cb = c_v.at[buf]
        ob = o_v.at[buf]
        nb = n_v.at[buf]
        sb = s_v.at[buf]

        def grp(g, carry, cb=cb, ob=ob, nb=nb, sb=sb):
            off = g * 16
            sl = pl.ds(off, 16)

            def dstep(d, accs, cb=cb, ob=ob, nb=nb, sl=sl):
                cd = cb[d, sl]
                new0 = accs[0] + cd * ob[d, sl]
                rest = tuple(
                    accs[1 + k] - nb[k, d, sl] * cd for k in range(_NNEG)
                )
                return (new0,) + rest

            zeros = tuple(jnp.zeros((16,), jnp.float32) for _ in range(1 + _NNEG))
            accs = lax.fori_loop(0, _DIM, dstep, zeros, unroll=8)
            for row in range(1 + _NNEG):
                sb[row, sl] = accs[row]
            return carry

        lax.fori_loop(0, _CH // 16, grp, 0)
        b0 = base + ci * _CH
        h = pltpu.async_copy(sb, out_hbm.at[:, pl.ds(b0, _CH)], osem.at[buf])
        out_pending[ci] = h
    for h in out_pending.values():
        h.wait()


def _log_sigmoid(x):
    return jnp.minimum(x, 0.0) - jnp.log1p(jnp.exp(-jnp.abs(x)))


def _finish_body(s_ref, out_ref):
    out_ref[0, 0] = jnp.sum(_log_sigmoid(s_ref[...]))


def kernel(center_word_vec, outside_word_vec, neg_word_vec):
    size, dim = center_word_vec.shape
    c_t = center_word_vec.T
    o_t = outside_word_vec.T
    n_t = jnp.transpose(neg_word_vec, (1, 2, 0))

    mesh = plsc.VectorSubcoreMesh(core_axis_name="c", subcore_axis_name="s")
    sc_scores = functools.partial(
        pl.kernel,
        out_type=jax.ShapeDtypeStruct((1 + _NNEG, size), jnp.float32),
        mesh=mesh,
        scratch_types=[
            pltpu.VMEM((2, _DIM, _CH), jnp.float32),
            pltpu.VMEM((2, _DIM, _CH), jnp.float32),
            pltpu.VMEM((2, _NNEG, _DIM, _CH), jnp.float32),
            pltpu.VMEM((2, 1 + _NNEG, _CH), jnp.float32),
            pltpu.SemaphoreType.DMA((2, 3)),
            pltpu.SemaphoreType.DMA((2,)),
        ],
        compiler_params=pltpu.CompilerParams(use_tc_tiling_on_sc=True),
    )(_sc_scores_body)
    scores = sc_scores(c_t, o_t, n_t)

    out = pl.pallas_call(
        _finish_body,
        out_specs=pl.BlockSpec(memory_space=pltpu.SMEM),
        out_shape=jax.ShapeDtypeStruct((1, 1), jnp.float32),
    )(scores)
    return -(out[0, 0] / size)


# speedup vs baseline: 1.1766x; 1.1766x over previous
import functools

import jax
import jax.numpy as jnp
from jax import lax
from jax.experimental import pallas as pl
from jax.experimental.pallas import tpu as pltpu
from jax.experimental.pallas import tpu_sc as plsc

_SIZE = 16384
_DIM = 64
_NNEG = 5
_NW = 32
_SC_SIZE = 4096
_TC_SIZE = _SIZE - _SC_SIZE
_PER_W = _SC_SIZE // _NW
_TC_B = 4096


def _sc_scores_body(ct_hbm, ot_hbm, nt_hbm, out_hbm, c_v, o_v, n_v, s_v):
    wid = lax.axis_index("s") * 2 + lax.axis_index("c")
    base = _TC_SIZE + wid * _PER_W
    sl_in = pl.ds(base, _PER_W)
    pltpu.sync_copy(ct_hbm.at[:, sl_in], c_v)
    pltpu.sync_copy(ot_hbm.at[:, sl_in], o_v)
    pltpu.sync_copy(nt_hbm.at[:, :, sl_in], n_v)

    def grp(g, carry):
        off = g * 16
        sl = pl.ds(off, 16)

        def dstep(d, accs):
            cd = c_v[d, sl]
            new0 = accs[0] + cd * o_v[d, sl]
            rest = tuple(accs[1 + k] - n_v[k, d, sl] * cd for k in range(_NNEG))
            return (new0,) + rest

        zeros = tuple(jnp.zeros((16,), jnp.float32) for _ in range(1 + _NNEG))
        accs = lax.fori_loop(0, _DIM, dstep, zeros, unroll=8)
        for row in range(1 + _NNEG):
            s_v[row, sl] = accs[row]
        return carry

    lax.fori_loop(0, _PER_W // 16, grp, 0)
    pltpu.sync_copy(s_v, out_hbm.at[:, pl.ds(wid * _PER_W, _PER_W)])


def _log_sigmoid(x):
    return jnp.minimum(x, 0.0) - jnp.log1p(jnp.exp(-jnp.abs(x)))


def _tc_body(c_ref, o_ref, n_ref, out_ref):
    c = c_ref[...]
    pos = jnp.sum(o_ref[...] * c, axis=0)
    acc = _log_sigmoid(pos)
    for k in range(_NNEG):
        s = jnp.sum(n_ref[k] * c, axis=0)
        acc = acc + _log_sigmoid(-s)
    partial = jnp.sum(acc)

    @pl.when(pl.program_id(0) == 0)
    def _():
        out_ref[0, 0] = 0.0

    out_ref[0, 0] += partial


def _finish_body(s_ref, p_ref, out_ref):
    out_ref[0, 0] = p_ref[0, 0] + jnp.sum(_log_sigmoid(s_ref[...]))


def kernel(center_word_vec, outside_word_vec, neg_word_vec):
    size, dim = center_word_vec.shape
    c_t = center_word_vec.T
    o_t = outside_word_vec.T
    n_t = jnp.transpose(neg_word_vec, (1, 2, 0))

    mesh = plsc.VectorSubcoreMesh(core_axis_name="c", subcore_axis_name="s")
    sc_scores = functools.partial(
        pl.kernel,
        out_type=jax.ShapeDtypeStruct((1 + _NNEG, _SC_SIZE), jnp.float32),
        mesh=mesh,
        scratch_types=[
            pltpu.VMEM((_DIM, _PER_W), jnp.float32),
            pltpu.VMEM((_DIM, _PER_W), jnp.float32),
            pltpu.VMEM((_NNEG, _DIM, _PER_W), jnp.float32),
            pltpu.VMEM((1 + _NNEG, _PER_W), jnp.float32),
        ],
        compiler_params=pltpu.CompilerParams(use_tc_tiling_on_sc=True),
    )(_sc_scores_body)
    scores = sc_scores(c_t, o_t, n_t)

    tc_partial = pl.pallas_call(
        _tc_body,
        grid=(_TC_SIZE // _TC_B,),
        in_specs=[
            pl.BlockSpec((dim, _TC_B), lambda i: (0, i)),
            pl.BlockSpec((dim, _TC_B), lambda i: (0, i)),
            pl.BlockSpec((_NNEG, dim, _TC_B), lambda i: (0, 0, i)),
        ],
        out_specs=pl.BlockSpec(memory_space=pltpu.SMEM),
        out_shape=jax.ShapeDtypeStruct((1, 1), jnp.float32),
    )(c_t, o_t, n_t)

    out = pl.pallas_call(
        _finish_body,
        in_specs=[
            pl.BlockSpec((1 + _NNEG, _SC_SIZE), lambda: (0, 0)),
            pl.BlockSpec(memory_space=pltpu.SMEM),
        ],
        out_specs=pl.BlockSpec(memory_space=pltpu.SMEM),
        out_shape=jax.ShapeDtypeStruct((1, 1), jnp.float32),
    )(scores, tc_partial)
    return -(out[0, 0] / size)


# speedup vs baseline: 2.6650x; 2.2651x over previous
import jax
import jax.numpy as jnp
from jax.experimental import pallas as pl
from jax.experimental.pallas import tpu as pltpu


def _log_sigmoid(x):
    return jnp.minimum(x, 0.0) - jnp.log1p(jnp.exp(-jnp.abs(x)))


def _body(c_ref, o_ref, n_ref, out_ref):
    c = c_ref[...]
    pos = jnp.sum(o_ref[...] * c, axis=0)
    acc = _log_sigmoid(pos)
    nneg = n_ref.shape[0]
    for k in range(nneg):
        s = jnp.sum(n_ref[k] * c, axis=0)
        acc = acc + _log_sigmoid(-s)
    partial = jnp.sum(acc)

    @pl.when(pl.program_id(0) == 0)
    def _():
        out_ref[0, 0] = 0.0

    out_ref[0, 0] += partial


def kernel(center_word_vec, outside_word_vec, neg_word_vec):
    size, dim = center_word_vec.shape
    nneg = neg_word_vec.shape[1]
    c_t = center_word_vec.T
    o_t = outside_word_vec.T
    n_t = jnp.transpose(neg_word_vec, (1, 2, 0))
    B = 8192
    grid = size // B
    out = pl.pallas_call(
        _body,
        grid=(grid,),
        in_specs=[
            pl.BlockSpec((dim, B), lambda i: (0, i)),
            pl.BlockSpec((dim, B), lambda i: (0, i)),
            pl.BlockSpec((nneg, dim, B), lambda i: (0, 0, i)),
        ],
        out_specs=pl.BlockSpec(memory_space=pltpu.SMEM),
        out_shape=jax.ShapeDtypeStruct((1, 1), jnp.float32),
    )(c_t, o_t, n_t)
    return -(out[0, 0] / size)


# speedup vs baseline: 2.8291x; 1.0616x over previous
import jax
import jax.numpy as jnp
from jax.experimental import pallas as pl
from jax.experimental.pallas import tpu as pltpu


def _log_sigmoid(x):
    return jnp.minimum(x, 0.0) - jnp.log1p(jnp.exp(-jnp.abs(x)))


def _body(c_ref, o_ref, n0, n1, n2, n3, n4, out_ref):
    c = c_ref[...]
    pos = jnp.sum(o_ref[...] * c, axis=0)
    acc = _log_sigmoid(pos)
    for n_ref in (n0, n1, n2, n3, n4):
        s = jnp.sum(n_ref[0] * c, axis=0)
        acc = acc + _log_sigmoid(-s)
    partial = jnp.sum(acc)

    @pl.when(pl.program_id(0) == 0)
    def _():
        out_ref[0, 0] = 0.0

    out_ref[0, 0] += partial


def kernel(center_word_vec, outside_word_vec, neg_word_vec):
    size, dim = center_word_vec.shape
    nneg = neg_word_vec.shape[1]
    c_t = center_word_vec.T
    o_t = outside_word_vec.T
    n_t = jnp.transpose(neg_word_vec, (1, 2, 0))
    B = 4096
    grid = size // B
    neg_specs = [
        pl.BlockSpec((1, dim, B), (lambda i, k=k: (k, 0, i))) for k in range(nneg)
    ]
    out = pl.pallas_call(
        _body,
        grid=(grid,),
        in_specs=[
            pl.BlockSpec((dim, B), lambda i: (0, i)),
            pl.BlockSpec((dim, B), lambda i: (0, i)),
            *neg_specs,
        ],
        out_specs=pl.BlockSpec(memory_space=pltpu.SMEM),
        out_shape=jax.ShapeDtypeStruct((1, 1), jnp.float32),
    )(c_t, o_t, *([n_t] * nneg))
    return -(out[0, 0] / size)
